# baseline (speedup 1.0000x reference)
import functools

import jax
import jax.numpy as jnp
from jax import lax
from jax.experimental import pallas as pl
from jax.experimental.pallas import tpu as pltpu
from jax.experimental.pallas import tpu_sc as plsc

NC = 2
NS = 16
LANES = 16
CH = 128

_BN_SCALE = 1.0 / (1.0 + 1e-5) ** 0.5


@functools.lru_cache(maxsize=None)
def _sc_deg(n_pad: int, nchunks: int):
    rpt = n_pad // NS
    mesh = plsc.VectorSubcoreMesh(core_axis_name="c", subcore_axis_name="s")

    @functools.partial(
        pl.kernel,
        out_type=jax.ShapeDtypeStruct((NC * n_pad,), jnp.float32),
        mesh=mesh,
        scratch_types=[
            pltpu.VMEM((nchunks, CH), jnp.int32),
            pltpu.VMEM((CH,), jnp.float32),
            pltpu.VMEM((rpt,), jnp.float32),
            pltpu.VMEM_SHARED((n_pad,), jnp.float32),
            pltpu.SemaphoreType.DMA,
        ],
        compiler_params=pltpu.CompilerParams(has_side_effects=True),
    )
    def k(row_hbm, out_hbm, ridx, ones, stag, acc, sem):
        c = lax.axis_index("c")
        s = lax.axis_index("s")
        w = c * NS + s

        def fill_ones(i, _):
            ones[pl.ds(i * LANES, LANES)] = jnp.full((LANES,), 1.0, jnp.float32)
            return 0
        lax.fori_loop(0, CH // LANES, fill_ones, 0)

        def fill_zero(i, _):
            stag[pl.ds(i * LANES, LANES)] = jnp.zeros((LANES,), jnp.float32)
            return 0
        lax.fori_loop(0, rpt // LANES, fill_zero, 0)
        pltpu.sync_copy(stag, acc.at[pl.ds(s * rpt, rpt)])
        plsc.subcore_barrier()

        pltpu.sync_copy(row_hbm.at[pl.ds(w * nchunks, nchunks)], ridx)

        def ebody(j, _):
            pltpu.async_copy(ones, acc.at[ridx.at[j]], sem, add=True)
            return 0
        lax.fori_loop(0, nchunks, ebody, 0)

        def edrain(j, _):
            pltpu.make_async_copy(ones, acc.at[ridx.at[j]], sem).wait()
            return 0
        lax.fori_loop(0, nchunks, edrain, 0)

        plsc.subcore_barrier()
        pltpu.sync_copy(acc.at[pl.ds(s * rpt, rpt)], stag)
        pltpu.sync_copy(stag, out_hbm.at[pl.ds(c * n_pad + s * rpt, rpt)])

    return k


NBUF = 2


@functools.lru_cache(maxsize=None)
def _sc_hop(n_pad: int, nchunks: int, dh: int):
    rpt = n_pad // NS
    mesh = plsc.VectorSubcoreMesh(core_axis_name="c", subcore_axis_name="s")

    @functools.partial(
        pl.kernel,
        out_type=jax.ShapeDtypeStruct((NC, n_pad, dh), jnp.float32),
        mesh=mesh,
        scratch_types=[
            pltpu.VMEM((nchunks, CH), jnp.int32),
            pltpu.VMEM((nchunks, CH), jnp.int32),
            pltpu.VMEM((NBUF, CH, dh), jnp.float32),
            pltpu.VMEM_SHARED((n_pad, dh), jnp.float32),
            pltpu.VMEM_SHARED((n_pad, dh), jnp.float32),
            pltpu.SemaphoreType.DMA,
            pltpu.SemaphoreType.DMA,
        ],
        compiler_params=pltpu.CompilerParams(use_tc_tiling_on_sc=False,
                                             has_side_effects=True),
    )
    def k(hs_hbm, row_hbm, col_hbm, out_hbm, ridx, cidx, rows, acc, hs_s,
          sem, ssem):
        c = lax.axis_index("c")
        s = lax.axis_index("s")
        w = c * NS + s
        vpr = dh // LANES

        def fill_zero(i, _):
            rows[0, i // vpr, pl.ds((i % vpr) * LANES, LANES)] = (
                jnp.zeros((LANES,), jnp.float32))
            return 0
        lax.fori_loop(0, CH * vpr, fill_zero, 0)

        nblk = rpt // CH
        for i in range(nblk):
            pltpu.async_copy(rows.at[0],
                             acc.at[pl.ds(s * rpt + i * CH, CH)], ssem)

        def hscopy(i, _):
            pltpu.sync_copy(hs_hbm.at[pl.ds(s * rpt + i * CH, CH)],
                            rows.at[1])
            pltpu.sync_copy(rows.at[1], hs_s.at[pl.ds(s * rpt + i * CH, CH)])
            return 0
        lax.fori_loop(0, nblk, hscopy, 0)
        for i in range(nblk):
            pltpu.make_async_copy(
                rows.at[0], acc.at[pl.ds(s * rpt + i * CH, CH)], ssem).wait()
        plsc.subcore_barrier()

        pltpu.sync_copy(row_hbm.at[pl.ds(w * nchunks, nchunks)], ridx)
        pltpu.sync_copy(col_hbm.at[pl.ds(w * nchunks, nchunks)], cidx)

        LAG = 1
        for b in range(NBUF):
            pltpu.async_copy(hs_s.at[ridx.at[b]], rows.at[b], sem)

        def ebody(g, _):
            for b in range(NBUF):
                j = g * NBUF + b
                pltpu.make_async_copy(
                    hs_s.at[ridx.at[j]], rows.at[b], sem).wait()
                pltpu.async_copy(rows.at[b], acc.at[cidx.at[j]], ssem,
                                 add=True)
                dj = j - LAG
                bd = (b - LAG) % NBUF

                @pl.when(dj >= 0)
                def _():
                    pltpu.make_async_copy(
                        rows.at[bd], acc.at[cidx.at[dj]], ssem).wait()
                    nj = dj + NBUF

                    @pl.when(nj < nchunks)
                    def _():
                        pltpu.async_copy(
                            hs_s.at[ridx.at[nj]], rows.at[bd], sem)
            return 0
        lax.fori_loop(0, nchunks // NBUF, ebody, 0)
        for t in range(LAG):
            j = nchunks - LAG + t
            pltpu.make_async_copy(
                rows.at[j % NBUF], acc.at[cidx.at[j]], ssem).wait()

        plsc.subcore_barrier()

        for i in range(nblk):
            b = i % 2
            pltpu.sync_copy(acc.at[pl.ds(s * rpt + i * CH, CH)], rows.at[b])
            if i >= 1:
                pltpu.make_async_copy(
                    rows.at[(i - 1) % 2],
                    out_hbm.at[c, pl.ds(s * rpt + (i - 1) * CH, CH)],
                    ssem).wait()
            pltpu.async_copy(rows.at[b],
                             out_hbm.at[c, pl.ds(s * rpt + i * CH, CH)], ssem)
        pltpu.make_async_copy(
            rows.at[(nblk - 1) % 2],
            out_hbm.at[c, pl.ds(s * rpt + (nblk - 1) * CH, CH)], ssem).wait()

    return k


def _tc_prep(x, w_feat, b_feat, deg_parts, n_pad):
    n, _ = x.shape
    dh = w_feat.shape[1]

    def body(x_ref, wf_ref, bf_ref, dp_ref, h_ref, hs_ref, dinv_ref):
        h = jnp.maximum(
            jnp.dot(x_ref[...], wf_ref[...],
                    preferred_element_type=jnp.float32) + bf_ref[...][None, :],
            0.0)
        deg = jnp.maximum(dp_ref[0, :] + dp_ref[1, :], 1.0)
        dinv_full = lax.rsqrt(deg).reshape(n_pad, 1)
        dinv = dinv_full[:n]
        h_ref[...] = h
        hs_ref[:n, :] = dinv * h
        hs_ref[n:, :] = jnp.zeros((n_pad - n, dh), jnp.float32)
        dinv_ref[...] = dinv

    return pl.pallas_call(
        body,
        out_shape=[
            jax.ShapeDtypeStruct((n, dh), jnp.float32),
            jax.ShapeDtypeStruct((n_pad, dh), jnp.float32),
            jax.ShapeDtypeStruct((n, 1), jnp.float32),
        ],
    )(x, w_feat, b_feat, deg_parts)


def _tc_mid(p0p1, dinv, n_pad):
    n = dinv.shape[0]
    dh = p0p1.shape[2]

    def body(p_ref, dinv_ref, hs_ref):
        raw = p_ref[0, :n, :] + p_ref[1, :n, :]
        d2 = dinv_ref[...] * dinv_ref[...]
        hs_ref[:n, :] = d2 * raw
        hs_ref[n:, :] = jnp.zeros((n_pad - n, dh), jnp.float32)

    return pl.pallas_call(
        body,
        out_shape=jax.ShapeDtypeStruct((n_pad, dh), jnp.float32),
    )(p0p1, dinv)


def _tc_layer(h, parts1, parts2, dinv, wl, bl, gl, betal, n_pad, final_w=None,
              final_b=None):
    n, dh = h.shape
    final = final_w is not None

    def body(h_ref, p1_ref, p2_ref, dinv_ref, wl_ref, bl_ref, gl_ref,
             betal_ref, *rest):
        if final:
            wc_ref, bc_ref, out_ref = rest
        else:
            hn_ref, hs_ref = rest
        dinv = dinv_ref[...]
        agg1 = dinv * (p1_ref[0, :n, :] + p1_ref[1, :n, :])
        agg2 = dinv * (p2_ref[0, :n, :] + p2_ref[1, :n, :])
        z = (jnp.dot(h_ref[...], wl_ref[:dh, :],
                     preferred_element_type=jnp.float32)
             + jnp.dot(agg1, wl_ref[dh:2 * dh, :],
                       preferred_element_type=jnp.float32)
             + jnp.dot(agg2, wl_ref[2 * dh:, :],
                       preferred_element_type=jnp.float32)
             + bl_ref[...][None, :])
        hn = jnp.maximum(gl_ref[...][None, :] * z * _BN_SCALE
                         + betal_ref[...][None, :], 0.0)
        if final:
            out_ref[...] = jnp.dot(hn, wc_ref[...],
                                   preferred_element_type=jnp.float32) \
                + bc_ref[...][None, :]
        else:
            hn_ref[...] = hn
            hs_ref[:n, :] = dinv * hn
            hs_ref[n:, :] = jnp.zeros((n_pad - n, dh), jnp.float32)

    if final:
        out_shape = jax.ShapeDtypeStruct((n, final_w.shape[1]), jnp.float32)
        return pl.pallas_call(body, out_shape=out_shape)(
            h, parts1, parts2, dinv, wl, bl, gl, betal, final_w, final_b)
    out_shape = [
        jax.ShapeDtypeStruct((n, dh), jnp.float32),
        jax.ShapeDtypeStruct((n_pad, dh), jnp.float32),
    ]
    return pl.pallas_call(body, out_shape=out_shape)(
        h, parts1, parts2, dinv, wl, bl, gl, betal)


def kernel(x, edge_index, W_feat, b_feat, W1, b1, g1, beta1, W2, b2, g2,
           beta2, Wc, bc):
    n, _ = x.shape
    e = edge_index.shape[1]
    dh = W_feat.shape[1]

    rpt = -(-n // NS)
    rpt = -(-rpt // CH) * CH
    n_pad = rpt * NS
    per_w = -(-e // (NC * NS))
    ew = -(-per_w // (8 * CH)) * (8 * CH)
    e_pad = ew * NC * NS
    nchunks = ew // CH

    row = edge_index[0]
    col = edge_index[1]
    pad = e_pad - e
    if pad:
        fill = jnp.full((pad,), n, jnp.int32)
        row_p = jnp.concatenate([row, fill])
        col_p = jnp.concatenate([col, fill])
    else:
        row_p, col_p = row, col
    row2d = row_p.reshape(NC * NS * nchunks, CH)
    col2d = col_p.reshape(NC * NS * nchunks, CH)

    deg_parts = _sc_deg(n_pad, nchunks)(row2d).reshape(NC, n_pad)
    h0, hs0, dinv = _tc_prep(x, W_feat, b_feat, deg_parts, n_pad)

    hop = _sc_hop(n_pad, nchunks, dh)
    h, hs = h0, hs0
    for li, (wl, bl, gl, betal) in enumerate([(W1, b1, g1, beta1),
                                              (W2, b2, g2, beta2)]):
        parts1 = hop(hs, row2d, col2d)
        hs_mid = _tc_mid(parts1, dinv, n_pad)
        parts2 = hop(hs_mid, row2d, col2d)
        if li == 0:
            h, hs = _tc_layer(h, parts1, parts2, dinv, wl, bl, gl, betal,
                              n_pad)
        else:
            return _tc_layer(h, parts1, parts2, dinv, wl, bl, gl, betal,
                             n_pad, final_w=Wc, final_b=bc)


# speedup vs baseline: 1.0189x; 1.0189x over previous
import functools

import jax
import jax.numpy as jnp
from jax import lax
from jax.experimental import pallas as pl
from jax.experimental.pallas import tpu as pltpu
from jax.experimental.pallas import tpu_sc as plsc

NC = 2
NS = 16
LANES = 16
CH = 128

_BN_SCALE = 1.0 / (1.0 + 1e-5) ** 0.5


@functools.lru_cache(maxsize=None)
def _sc_deg(n_pad: int, nchunks: int):
    rpt = n_pad // NS
    mesh = plsc.VectorSubcoreMesh(core_axis_name="c", subcore_axis_name="s")

    @functools.partial(
        pl.kernel,
        out_type=jax.ShapeDtypeStruct((NC * n_pad,), jnp.float32),
        mesh=mesh,
        scratch_types=[
            pltpu.VMEM((nchunks, CH), jnp.int32),
            pltpu.VMEM((CH,), jnp.float32),
            pltpu.VMEM((rpt,), jnp.float32),
            pltpu.VMEM_SHARED((n_pad,), jnp.float32),
            pltpu.SemaphoreType.DMA,
        ],
        compiler_params=pltpu.CompilerParams(has_side_effects=True),
    )
    def k(row_hbm, out_hbm, ridx, ones, stag, acc, sem):
        c = lax.axis_index("c")
        s = lax.axis_index("s")
        w = c * NS + s

        def fill_ones(i, _):
            ones[pl.ds(i * LANES, LANES)] = jnp.full((LANES,), 1.0, jnp.float32)
            return 0
        lax.fori_loop(0, CH // LANES, fill_ones, 0)

        def fill_zero(i, _):
            stag[pl.ds(i * LANES, LANES)] = jnp.zeros((LANES,), jnp.float32)
            return 0
        lax.fori_loop(0, rpt // LANES, fill_zero, 0)
        pltpu.sync_copy(stag, acc.at[pl.ds(s * rpt, rpt)])
        plsc.subcore_barrier()

        pltpu.sync_copy(row_hbm.at[pl.ds(w * nchunks, nchunks)], ridx)

        def ebody(j, _):
            pltpu.async_copy(ones, acc.at[ridx.at[j]], sem, add=True)
            return 0
        lax.fori_loop(0, nchunks, ebody, 0)

        def edrain(j, _):
            pltpu.make_async_copy(ones, acc.at[ridx.at[j]], sem).wait()
            return 0
        lax.fori_loop(0, nchunks, edrain, 0)

        plsc.subcore_barrier()
        pltpu.sync_copy(acc.at[pl.ds(s * rpt, rpt)], stag)
        pltpu.sync_copy(stag, out_hbm.at[pl.ds(c * n_pad + s * rpt, rpt)])

    return k


NBUF = 2


@functools.lru_cache(maxsize=None)
def _sc_hop(n_pad: int, nchunks: int, dh: int, kparts: int):
    rpt = n_pad // NS
    mesh = plsc.VectorSubcoreMesh(core_axis_name="c", subcore_axis_name="s")

    @functools.partial(
        pl.kernel,
        out_type=jax.ShapeDtypeStruct((NC, n_pad, dh), jnp.float32),
        mesh=mesh,
        scratch_types=[
            pltpu.VMEM((nchunks, CH), jnp.int32),
            pltpu.VMEM((nchunks, CH), jnp.int32),
            pltpu.VMEM((NBUF, CH, dh), jnp.float32),
            pltpu.VMEM((CH, dh), jnp.float32),
            pltpu.VMEM((rpt,), jnp.float32),
            pltpu.VMEM_SHARED((n_pad, dh), jnp.float32),
            pltpu.VMEM_SHARED((n_pad, dh), jnp.float32),
            pltpu.SemaphoreType.DMA,
            pltpu.SemaphoreType.DMA,
        ],
        compiler_params=pltpu.CompilerParams(use_tc_tiling_on_sc=False,
                                             has_side_effects=True,
                                             needs_layout_passes=False),
    )
    def k(p_hbm, scale_hbm, row_hbm, col_hbm, out_hbm, ridx, cidx, rows,
          pbuf, svec, acc, hs_s, sem, ssem):
        c = lax.axis_index("c")
        s = lax.axis_index("s")
        w = c * NS + s
        vpr = dh // LANES

        def fill_zero(i, _):
            rows[0, i // vpr, pl.ds((i % vpr) * LANES, LANES)] = (
                jnp.zeros((LANES,), jnp.float32))
            return 0
        lax.fori_loop(0, CH * vpr, fill_zero, 0)

        nblk = rpt // CH
        for i in range(nblk):
            pltpu.async_copy(rows.at[0],
                             acc.at[pl.ds(s * rpt + i * CH, CH)], ssem)

        pltpu.sync_copy(scale_hbm.at[pl.ds(s * rpt, rpt)], svec)

        def hscopy(i, _):
            pltpu.sync_copy(p_hbm.at[0, pl.ds(s * rpt + i * CH, CH)],
                            rows.at[1])
            if kparts == 2:
                pltpu.sync_copy(p_hbm.at[1, pl.ds(s * rpt + i * CH, CH)],
                                pbuf)

            def scale_row(r, _):
                v = plsc.load_gather(
                    svec, [jnp.full((LANES,), i * CH + r, jnp.int32)])
                for q in range(vpr):
                    cur = rows[1, r, pl.ds(q * LANES, LANES)]
                    if kparts == 2:
                        cur = cur + pbuf[r, pl.ds(q * LANES, LANES)]
                    rows[1, r, pl.ds(q * LANES, LANES)] = cur * v
                return 0
            lax.fori_loop(0, CH, scale_row, 0)
            pltpu.sync_copy(rows.at[1], hs_s.at[pl.ds(s * rpt + i * CH, CH)])
            return 0
        lax.fori_loop(0, nblk, hscopy, 0)
        for i in range(nblk):
            pltpu.make_async_copy(
                rows.at[0], acc.at[pl.ds(s * rpt + i * CH, CH)], ssem).wait()
        plsc.subcore_barrier()

        pltpu.sync_copy(row_hbm.at[pl.ds(w * nchunks, nchunks)], ridx)
        pltpu.sync_copy(col_hbm.at[pl.ds(w * nchunks, nchunks)], cidx)

        LAG = 1
        for b in range(NBUF):
            pltpu.async_copy(hs_s.at[ridx.at[b]], rows.at[b], sem)

        def ebody(g, _):
            for b in range(NBUF):
                j = g * NBUF + b
                pltpu.make_async_copy(
                    hs_s.at[ridx.at[j]], rows.at[b], sem).wait()
                pltpu.async_copy(rows.at[b], acc.at[cidx.at[j]], ssem,
                                 add=True)
                dj = j - LAG
                bd = (b - LAG) % NBUF

                @pl.when(dj >= 0)
                def _():
                    pltpu.make_async_copy(
                        rows.at[bd], acc.at[cidx.at[dj]], ssem).wait()
                    nj = dj + NBUF

                    @pl.when(nj < nchunks)
                    def _():
                        pltpu.async_copy(
                            hs_s.at[ridx.at[nj]], rows.at[bd], sem)
            return 0
        lax.fori_loop(0, nchunks // NBUF, ebody, 0)
        for t in range(LAG):
            j = nchunks - LAG + t
            pltpu.make_async_copy(
                rows.at[j % NBUF], acc.at[cidx.at[j]], ssem).wait()

        plsc.subcore_barrier()

        for i in range(nblk):
            b = i % 2
            pltpu.sync_copy(acc.at[pl.ds(s * rpt + i * CH, CH)], rows.at[b])
            if i >= 1:
                pltpu.make_async_copy(
                    rows.at[(i - 1) % 2],
                    out_hbm.at[c, pl.ds(s * rpt + (i - 1) * CH, CH)],
                    ssem).wait()
            pltpu.async_copy(rows.at[b],
                             out_hbm.at[c, pl.ds(s * rpt + i * CH, CH)], ssem)
        pltpu.make_async_copy(
            rows.at[(nblk - 1) % 2],
            out_hbm.at[c, pl.ds(s * rpt + (nblk - 1) * CH, CH)], ssem).wait()

    return k


def _tc_prep(x, w_feat, b_feat, deg_parts, n_pad):
    n, _ = x.shape
    dh = w_feat.shape[1]

    def body(x_ref, wf_ref, bf_ref, dp_ref, h_ref, dinv_ref, d1_ref, d2_ref):
        h = jnp.maximum(
            jnp.dot(x_ref[...], wf_ref[...],
                    preferred_element_type=jnp.float32) + bf_ref[...][None, :],
            0.0)
        deg = jnp.maximum(dp_ref[0, :] + dp_ref[1, :], 1.0)
        dinv_full = lax.rsqrt(deg).reshape(n_pad, 1)
        dinv = dinv_full[:n]
        h_ref[:n, :] = h
        h_ref[n:, :] = jnp.zeros((n_pad - n, dh), jnp.float32)
        dinv_ref[...] = dinv
        zt = jnp.zeros((n_pad - n,), jnp.float32)
        d1_ref[...] = jnp.concatenate([dinv[:, 0], zt])
        d2_ref[...] = jnp.concatenate([dinv[:, 0] * dinv[:, 0], zt])

    return pl.pallas_call(
        body,
        out_shape=[
            jax.ShapeDtypeStruct((n_pad, dh), jnp.float32),
            jax.ShapeDtypeStruct((n, 1), jnp.float32),
            jax.ShapeDtypeStruct((n_pad,), jnp.float32),
            jax.ShapeDtypeStruct((n_pad,), jnp.float32),
        ],
    )(x, w_feat, b_feat, deg_parts)


def _tc_layer(h_pad, parts1, parts2, dinv, wl, bl, gl, betal, n_pad,
              final_w=None, final_b=None):
    n = dinv.shape[0]
    dh = h_pad.shape[1]
    final = final_w is not None

    def body(h_ref, p1_ref, p2_ref, dinv_ref, wl_ref, bl_ref, gl_ref,
             betal_ref, *rest):
        if final:
            wc_ref, bc_ref, out_ref = rest
        else:
            (hn_ref,) = rest
        dinv = dinv_ref[...]
        agg1 = dinv * (p1_ref[0, :n, :] + p1_ref[1, :n, :])
        agg2 = dinv * (p2_ref[0, :n, :] + p2_ref[1, :n, :])
        z = (jnp.dot(h_ref[:n, :], wl_ref[:dh, :],
                     preferred_element_type=jnp.float32)
             + jnp.dot(agg1, wl_ref[dh:2 * dh, :],
                       preferred_element_type=jnp.float32)
             + jnp.dot(agg2, wl_ref[2 * dh:, :],
                       preferred_element_type=jnp.float32)
             + bl_ref[...][None, :])
        hn = jnp.maximum(gl_ref[...][None, :] * z * _BN_SCALE
                         + betal_ref[...][None, :], 0.0)
        if final:
            out_ref[...] = jnp.dot(hn, wc_ref[...],
                                   preferred_element_type=jnp.float32) \
                + bc_ref[...][None, :]
        else:
            hn_ref[:n, :] = hn
            hn_ref[n:, :] = jnp.zeros((n_pad - n, dh), jnp.float32)

    if final:
        out_shape = jax.ShapeDtypeStruct((n, final_w.shape[1]), jnp.float32)
        return pl.pallas_call(body, out_shape=out_shape)(
            h_pad, parts1, parts2, dinv, wl, bl, gl, betal, final_w, final_b)
    out_shape = jax.ShapeDtypeStruct((n_pad, dh), jnp.float32)
    return pl.pallas_call(body, out_shape=out_shape)(
        h_pad, parts1, parts2, dinv, wl, bl, gl, betal)


def kernel(x, edge_index, W_feat, b_feat, W1, b1, g1, beta1, W2, b2, g2,
           beta2, Wc, bc):
    n, _ = x.shape
    e = edge_index.shape[1]
    dh = W_feat.shape[1]

    rpt = -(-n // NS)
    rpt = -(-rpt // CH) * CH
    n_pad = rpt * NS
    per_w = -(-e // (NC * NS))
    ew = -(-per_w // (8 * CH)) * (8 * CH)
    e_pad = ew * NC * NS
    nchunks = ew // CH

    row = edge_index[0]
    col = edge_index[1]
    pad = e_pad - e
    if pad:
        fill = jnp.full((pad,), n, jnp.int32)
        row_p = jnp.concatenate([row, fill])
        col_p = jnp.concatenate([col, fill])
    else:
        row_p, col_p = row, col
    row2d = row_p.reshape(NC * NS * nchunks, CH)
    col2d = col_p.reshape(NC * NS * nchunks, CH)

    deg_parts = _sc_deg(n_pad, nchunks)(row2d).reshape(NC, n_pad)
    h, dinv, dinv_pad, dinv2_pad = _tc_prep(x, W_feat, b_feat, deg_parts,
                                            n_pad)

    hop1 = _sc_hop(n_pad, nchunks, dh, 1)
    hop2 = _sc_hop(n_pad, nchunks, dh, 2)
    for li, (wl, bl, gl, betal) in enumerate([(W1, b1, g1, beta1),
                                              (W2, b2, g2, beta2)]):
        parts1 = hop1(h.reshape(1, n_pad, dh), dinv_pad, row2d, col2d)
        parts2 = hop2(parts1, dinv2_pad, row2d, col2d)
        if li == 0:
            h = _tc_layer(h, parts1, parts2, dinv, wl, bl, gl, betal, n_pad)
        else:
            return _tc_layer(h, parts1, parts2, dinv, wl, bl, gl, betal,
                             n_pad, final_w=Wc, final_b=bc)


# speedup vs baseline: 1.0324x; 1.0133x over previous
import functools

import jax
import jax.numpy as jnp
from jax import lax
from jax.experimental import pallas as pl
from jax.experimental.pallas import tpu as pltpu
from jax.experimental.pallas import tpu_sc as plsc

NC = 2
NS = 16
LANES = 16
CH = 128

_BN_SCALE = 1.0 / (1.0 + 1e-5) ** 0.5


@functools.lru_cache(maxsize=None)
def _sc_deg(n_pad: int, nchunks: int):
    rpt = n_pad // NS
    mesh = plsc.VectorSubcoreMesh(core_axis_name="c", subcore_axis_name="s")

    @functools.partial(
        pl.kernel,
        out_type=jax.ShapeDtypeStruct((NC * n_pad,), jnp.float32),
        mesh=mesh,
        scratch_types=[
            pltpu.VMEM((nchunks, CH), jnp.int32),
            pltpu.VMEM((CH,), jnp.float32),
            pltpu.VMEM((rpt,), jnp.float32),
            pltpu.VMEM_SHARED((n_pad,), jnp.float32),
            pltpu.SemaphoreType.DMA,
        ],
        compiler_params=pltpu.CompilerParams(has_side_effects=True),
    )
    def k(row_hbm, out_hbm, ridx, ones, stag, acc, sem):
        c = lax.axis_index("c")
        s = lax.axis_index("s")
        w = c * NS + s

        def fill_ones(i, _):
            ones[pl.ds(i * LANES, LANES)] = jnp.full((LANES,), 1.0, jnp.float32)
            return 0
        lax.fori_loop(0, CH // LANES, fill_ones, 0)

        def fill_zero(i, _):
            stag[pl.ds(i * LANES, LANES)] = jnp.zeros((LANES,), jnp.float32)
            return 0
        lax.fori_loop(0, rpt // LANES, fill_zero, 0)
        pltpu.sync_copy(stag, acc.at[pl.ds(s * rpt, rpt)])
        plsc.subcore_barrier()

        pltpu.sync_copy(row_hbm.at[pl.ds(w * nchunks, nchunks)], ridx)

        def ebody(j, _):
            pltpu.async_copy(ones, acc.at[ridx.at[j]], sem, add=True)
            return 0
        lax.fori_loop(0, nchunks, ebody, 0)

        def edrain(j, _):
            pltpu.make_async_copy(ones, acc.at[ridx.at[j]], sem).wait()
            return 0
        lax.fori_loop(0, nchunks, edrain, 0)

        plsc.subcore_barrier()
        pltpu.sync_copy(acc.at[pl.ds(s * rpt, rpt)], stag)
        pltpu.sync_copy(stag, out_hbm.at[pl.ds(c * n_pad + s * rpt, rpt)])

    return k


NBUF = 2


@functools.lru_cache(maxsize=None)
def _sc_hop(n_pad: int, nchunks: int, dh: int, kparts: int):
    rpt = n_pad // NS
    mesh = plsc.VectorSubcoreMesh(core_axis_name="c", subcore_axis_name="s")

    @functools.partial(
        pl.kernel,
        out_type=jax.ShapeDtypeStruct((NC, n_pad, dh), jnp.float32),
        mesh=mesh,
        scratch_types=[
            pltpu.VMEM((nchunks, CH), jnp.int32),
            pltpu.VMEM((nchunks, CH), jnp.int32),
            pltpu.VMEM((NBUF, CH, dh), jnp.float32),
            pltpu.VMEM((CH, dh), jnp.float32),
            pltpu.VMEM((rpt,), jnp.float32),
            pltpu.VMEM_SHARED((n_pad, dh), jnp.float32),
            pltpu.VMEM_SHARED((n_pad, dh), jnp.float32),
            pltpu.SemaphoreType.DMA,
            pltpu.SemaphoreType.DMA,
        ],
        compiler_params=pltpu.CompilerParams(use_tc_tiling_on_sc=False,
                                             has_side_effects=True,
                                             needs_layout_passes=False),
    )
    def k(p_hbm, scale_hbm, row_hbm, col_hbm, out_hbm, ridx, cidx, rows,
          pbuf, svec, acc, hs_s, sem, ssem):
        c = lax.axis_index("c")
        s = lax.axis_index("s")
        w = c * NS + s
        vpr = dh // LANES

        def fill_zero(i, _):
            rows[0, i // vpr, pl.ds((i % vpr) * LANES, LANES)] = (
                jnp.zeros((LANES,), jnp.float32))
            return 0
        lax.fori_loop(0, CH * vpr, fill_zero, 0)

        nblk = rpt // CH
        for i in range(nblk):
            pltpu.async_copy(rows.at[0],
                             acc.at[pl.ds(s * rpt + i * CH, CH)], ssem)

        pltpu.sync_copy(scale_hbm.at[pl.ds(s * rpt, rpt)], svec)

        def hscopy(i, _):
            pltpu.sync_copy(p_hbm.at[0, pl.ds(s * rpt + i * CH, CH)],
                            rows.at[1])
            if kparts == 2:
                pltpu.sync_copy(p_hbm.at[1, pl.ds(s * rpt + i * CH, CH)],
                                pbuf)

            def scale_row(r, _):
                v = plsc.load_gather(
                    svec, [jnp.full((LANES,), i * CH + r, jnp.int32)])
                for q in range(vpr):
                    cur = rows[1, r, pl.ds(q * LANES, LANES)]
                    if kparts == 2:
                        cur = cur + pbuf[r, pl.ds(q * LANES, LANES)]
                    rows[1, r, pl.ds(q * LANES, LANES)] = cur * v
                return 0
            lax.fori_loop(0, CH, scale_row, 0)
            pltpu.sync_copy(rows.at[1], hs_s.at[pl.ds(s * rpt + i * CH, CH)])
            return 0
        lax.fori_loop(0, nblk, hscopy, 0)
        for i in range(nblk):
            pltpu.make_async_copy(
                rows.at[0], acc.at[pl.ds(s * rpt + i * CH, CH)], ssem).wait()
        plsc.subcore_barrier()

        pltpu.sync_copy(row_hbm.at[pl.ds(w * nchunks, nchunks)], ridx)
        pltpu.sync_copy(col_hbm.at[pl.ds(w * nchunks, nchunks)], cidx)

        NB = 3
        LAG = 2
        bufs = [rows.at[0], rows.at[1], pbuf]
        ntail = nchunks % NB
        nloop = nchunks - ntail
        for b in range(min(NB, nchunks)):
            pltpu.async_copy(hs_s.at[ridx.at[b]], bufs[b], sem)

        def ebody(g, _):
            for b in range(NB):
                j = g * NB + b
                pltpu.make_async_copy(
                    hs_s.at[ridx.at[j]], bufs[b], sem).wait()
                pltpu.async_copy(bufs[b], acc.at[cidx.at[j]], ssem,
                                 add=True)
                dj = j - LAG
                bd = (b - LAG) % NB

                @pl.when(dj >= 0)
                def _():
                    pltpu.make_async_copy(
                        bufs[bd], acc.at[cidx.at[dj]], ssem).wait()
                    nj = dj + NB

                    @pl.when(nj < nchunks)
                    def _():
                        pltpu.async_copy(
                            hs_s.at[ridx.at[nj]], bufs[bd], sem)
            return 0
        lax.fori_loop(0, nloop // NB, ebody, 0)
        for t in range(ntail):
            j = nloop + t
            b = j % NB
            pltpu.make_async_copy(hs_s.at[ridx.at[j]], bufs[b], sem).wait()
            pltpu.async_copy(bufs[b], acc.at[cidx.at[j]], ssem, add=True)
            dj = j - LAG
            if dj >= 0:
                pltpu.make_async_copy(
                    bufs[dj % NB], acc.at[cidx.at[dj]], ssem).wait()
                nj = dj + NB
                if nj < nchunks:
                    pltpu.async_copy(hs_s.at[ridx.at[nj]], bufs[nj % NB], sem)
        for t in range(min(LAG, nchunks)):
            j = nchunks - min(LAG, nchunks) + t
            pltpu.make_async_copy(
                bufs[j % NB], acc.at[cidx.at[j]], ssem).wait()

        plsc.subcore_barrier()

        for i in range(nblk):
            b = i % 2
            pltpu.sync_copy(acc.at[pl.ds(s * rpt + i * CH, CH)], rows.at[b])
            if i >= 1:
                pltpu.make_async_copy(
                    rows.at[(i - 1) % 2],
                    out_hbm.at[c, pl.ds(s * rpt + (i - 1) * CH, CH)],
                    ssem).wait()
            pltpu.async_copy(rows.at[b],
                             out_hbm.at[c, pl.ds(s * rpt + i * CH, CH)], ssem)
        pltpu.make_async_copy(
            rows.at[(nblk - 1) % 2],
            out_hbm.at[c, pl.ds(s * rpt + (nblk - 1) * CH, CH)], ssem).wait()

    return k


def _tc_prep(x, w_feat, b_feat, deg_parts, n_pad):
    n, _ = x.shape
    dh = w_feat.shape[1]

    def body(x_ref, wf_ref, bf_ref, dp_ref, h_ref, dinv_ref, d1_ref, d2_ref):
        h = jnp.maximum(
            jnp.dot(x_ref[...], wf_ref[...],
                    preferred_element_type=jnp.float32) + bf_ref[...][None, :],
            0.0)
        deg = jnp.maximum(dp_ref[0, :] + dp_ref[1, :], 1.0)
        dinv_full = lax.rsqrt(deg).reshape(n_pad, 1)
        dinv = dinv_full[:n]
        h_ref[:n, :] = h
        h_ref[n:, :] = jnp.zeros((n_pad - n, dh), jnp.float32)
        dinv_ref[...] = dinv
        zt = jnp.zeros((n_pad - n,), jnp.float32)
        d1_ref[...] = jnp.concatenate([dinv[:, 0], zt])
        d2_ref[...] = jnp.concatenate([dinv[:, 0] * dinv[:, 0], zt])

    return pl.pallas_call(
        body,
        out_shape=[
            jax.ShapeDtypeStruct((n_pad, dh), jnp.float32),
            jax.ShapeDtypeStruct((n, 1), jnp.float32),
            jax.ShapeDtypeStruct((n_pad,), jnp.float32),
            jax.ShapeDtypeStruct((n_pad,), jnp.float32),
        ],
    )(x, w_feat, b_feat, deg_parts)


def _tc_layer(h_pad, parts1, parts2, dinv, wl, bl, gl, betal, n_pad,
              final_w=None, final_b=None):
    n = dinv.shape[0]
    dh = h_pad.shape[1]
    final = final_w is not None

    def body(h_ref, p1_ref, p2_ref, dinv_ref, wl_ref, bl_ref, gl_ref,
             betal_ref, *rest):
        if final:
            wc_ref, bc_ref, out_ref = rest
        else:
            (hn_ref,) = rest
        dinv = dinv_ref[...]
        agg1 = dinv * (p1_ref[0, :n, :] + p1_ref[1, :n, :])
        agg2 = dinv * (p2_ref[0, :n, :] + p2_ref[1, :n, :])
        z = (jnp.dot(h_ref[:n, :], wl_ref[:dh, :],
                     preferred_element_type=jnp.float32)
             + jnp.dot(agg1, wl_ref[dh:2 * dh, :],
                       preferred_element_type=jnp.float32)
             + jnp.dot(agg2, wl_ref[2 * dh:, :],
                       preferred_element_type=jnp.float32)
             + bl_ref[...][None, :])
        hn = jnp.maximum(gl_ref[...][None, :] * z * _BN_SCALE
                         + betal_ref[...][None, :], 0.0)
        if final:
            out_ref[...] = jnp.dot(hn, wc_ref[...],
                                   preferred_element_type=jnp.float32) \
                + bc_ref[...][None, :]
        else:
            hn_ref[:n, :] = hn
            hn_ref[n:, :] = jnp.zeros((n_pad - n, dh), jnp.float32)

    if final:
        out_shape = jax.ShapeDtypeStruct((n, final_w.shape[1]), jnp.float32)
        return pl.pallas_call(body, out_shape=out_shape)(
            h_pad, parts1, parts2, dinv, wl, bl, gl, betal, final_w, final_b)
    out_shape = jax.ShapeDtypeStruct((n_pad, dh), jnp.float32)
    return pl.pallas_call(body, out_shape=out_shape)(
        h_pad, parts1, parts2, dinv, wl, bl, gl, betal)


def kernel(x, edge_index, W_feat, b_feat, W1, b1, g1, beta1, W2, b2, g2,
           beta2, Wc, bc):
    n, _ = x.shape
    e = edge_index.shape[1]
    dh = W_feat.shape[1]

    rpt = -(-n // NS)
    rpt = -(-rpt // CH) * CH
    n_pad = rpt * NS
    per_w = -(-e // (NC * NS))
    ew = -(-per_w // (8 * CH)) * (8 * CH)
    e_pad = ew * NC * NS
    nchunks = ew // CH

    row = edge_index[0]
    col = edge_index[1]
    pad = e_pad - e
    if pad:
        fill = jnp.full((pad,), n, jnp.int32)
        row_p = jnp.concatenate([row, fill])
        col_p = jnp.concatenate([col, fill])
    else:
        row_p, col_p = row, col
    row2d = row_p.reshape(NC * NS * nchunks, CH)
    col2d = col_p.reshape(NC * NS * nchunks, CH)

    deg_parts = _sc_deg(n_pad, nchunks)(row2d).reshape(NC, n_pad)
    h, dinv, dinv_pad, dinv2_pad = _tc_prep(x, W_feat, b_feat, deg_parts,
                                            n_pad)

    hop1 = _sc_hop(n_pad, nchunks, dh, 1)
    hop2 = _sc_hop(n_pad, nchunks, dh, 2)
    for li, (wl, bl, gl, betal) in enumerate([(W1, b1, g1, beta1),
                                              (W2, b2, g2, beta2)]):
        parts1 = hop1(h.reshape(1, n_pad, dh), dinv_pad, row2d, col2d)
        parts2 = hop2(parts1, dinv2_pad, row2d, col2d)
        if li == 0:
            h = _tc_layer(h, parts1, parts2, dinv, wl, bl, gl, betal, n_pad)
        else:
            return _tc_layer(h, parts1, parts2, dinv, wl, bl, gl, betal,
                             n_pad, final_w=Wc, final_b=bc)


# speedup vs baseline: 1.0628x; 1.0294x over previous
import functools

import jax
import jax.numpy as jnp
from jax import lax
from jax.experimental import pallas as pl
from jax.experimental.pallas import tpu as pltpu
from jax.experimental.pallas import tpu_sc as plsc

NC = 2
NS = 16
LANES = 16
CH = 128

_BN_SCALE = 1.0 / (1.0 + 1e-5) ** 0.5


@functools.lru_cache(maxsize=None)
def _sc_deg(n_pad: int, nchunks: int, n: int):
    rpt = n_pad // NS
    ncpt = NC * nchunks
    mesh = plsc.VectorSubcoreMesh(core_axis_name="c", subcore_axis_name="s")

    @functools.partial(
        pl.kernel,
        out_type=jax.ShapeDtypeStruct((2 * n_pad,), jnp.float32),
        mesh=mesh,
        scratch_types=[
            pltpu.VMEM((NC * nchunks, CH), jnp.int32),
            pltpu.VMEM((CH,), jnp.float32),
            pltpu.VMEM((rpt,), jnp.float32),
            pltpu.VMEM((rpt,), jnp.float32),
            pltpu.VMEM((rpt,), jnp.float32),
            pltpu.VMEM_SHARED((n_pad,), jnp.float32),
            pltpu.SemaphoreType.DMA,
        ],
        compiler_params=pltpu.CompilerParams(has_side_effects=True,
                                             needs_layout_passes=False),
    )
    def k(row_hbm, out_hbm, ridx, ones, stag, d1, d2, acc, sem):
        c = lax.axis_index("c")
        s = lax.axis_index("s")

        @pl.when(c == 0)
        def _core0():
            def fill_ones(i, _):
                ones[pl.ds(i * LANES, LANES)] = jnp.full((LANES,), 1.0,
                                                         jnp.float32)
                return 0
            lax.fori_loop(0, CH // LANES, fill_ones, 0)

            def fill_zero(i, _):
                stag[pl.ds(i * LANES, LANES)] = jnp.zeros((LANES,),
                                                          jnp.float32)
                return 0
            lax.fori_loop(0, rpt // LANES, fill_zero, 0)
            pltpu.sync_copy(stag, acc.at[pl.ds(s * rpt, rpt)])
            plsc.subcore_barrier()

            pltpu.sync_copy(row_hbm.at[pl.ds(s * ncpt, ncpt)], ridx)

            def ebody(j, _):
                pltpu.async_copy(ones, acc.at[ridx.at[j]], sem, add=True)
                return 0
            lax.fori_loop(0, ncpt, ebody, 0)

            def edrain(j, _):
                pltpu.make_async_copy(ones, acc.at[ridx.at[j]], sem).wait()
                return 0
            lax.fori_loop(0, ncpt, edrain, 0)

            plsc.subcore_barrier()
            pltpu.sync_copy(acc.at[pl.ds(s * rpt, rpt)], stag)

            def rsq(i, _):
                x = jnp.maximum(stag[pl.ds(i * LANES, LANES)], 1.0)
                seed = jnp.full((LANES,), 0x5F3759DF, jnp.int32) - (
                    plsc.bitcast(x, jnp.int32) >> 1)
                y = plsc.bitcast(seed, jnp.float32)
                for _it in range(3):
                    y = y * (1.5 - 0.5 * x * y * y)
                gidx = s * rpt + i * LANES + lax.iota(jnp.int32, LANES)
                y = jnp.where(gidx < n, y, 0.0)
                d1[pl.ds(i * LANES, LANES)] = y
                d2[pl.ds(i * LANES, LANES)] = y * y
                return 0
            lax.fori_loop(0, rpt // LANES, rsq, 0)
            pltpu.sync_copy(d1, out_hbm.at[pl.ds(s * rpt, rpt)])
            pltpu.sync_copy(d2, out_hbm.at[pl.ds(n_pad + s * rpt, rpt)])

    return k


NBUF = 2


@functools.lru_cache(maxsize=None)
def _sc_hop(n_pad: int, nchunks: int, dh: int, kparts: int):
    rpt = n_pad // NS
    mesh = plsc.VectorSubcoreMesh(core_axis_name="c", subcore_axis_name="s")

    @functools.partial(
        pl.kernel,
        out_type=jax.ShapeDtypeStruct((NC, n_pad, dh), jnp.float32),
        mesh=mesh,
        scratch_types=[
            pltpu.VMEM((nchunks, CH), jnp.int32),
            pltpu.VMEM((nchunks, CH), jnp.int32),
            pltpu.VMEM((NBUF, CH, dh), jnp.float32),
            pltpu.VMEM((CH, dh), jnp.float32),
            pltpu.VMEM((rpt,), jnp.float32),
            pltpu.VMEM_SHARED((n_pad, dh), jnp.float32),
            pltpu.VMEM_SHARED((n_pad, dh), jnp.float32),
            pltpu.SemaphoreType.DMA,
            pltpu.SemaphoreType.DMA,
        ],
        compiler_params=pltpu.CompilerParams(use_tc_tiling_on_sc=False,
                                             has_side_effects=True,
                                             needs_layout_passes=False),
    )
    def k(p_hbm, scale_hbm, row_hbm, col_hbm, out_hbm, ridx, cidx, rows,
          pbuf, svec, acc, hs_s, sem, ssem):
        c = lax.axis_index("c")
        s = lax.axis_index("s")
        w = c * NS + s
        vpr = dh // LANES

        def fill_zero(i, _):
            rows[0, i // vpr, pl.ds((i % vpr) * LANES, LANES)] = (
                jnp.zeros((LANES,), jnp.float32))
            return 0
        lax.fori_loop(0, CH * vpr, fill_zero, 0)

        nblk = rpt // CH
        for i in range(nblk):
            pltpu.async_copy(rows.at[0],
                             acc.at[pl.ds(s * rpt + i * CH, CH)], ssem)

        pltpu.sync_copy(scale_hbm.at[pl.ds(s * rpt, rpt)], svec)

        def hscopy(i, _):
            pltpu.sync_copy(p_hbm.at[0, pl.ds(s * rpt + i * CH, CH)],
                            rows.at[1])
            if kparts == 2:
                pltpu.sync_copy(p_hbm.at[1, pl.ds(s * rpt + i * CH, CH)],
                                pbuf)

            def scale_row(r, _):
                v = plsc.load_gather(
                    svec, [jnp.full((LANES,), i * CH + r, jnp.int32)])
                for q in range(vpr):
                    cur = rows[1, r, pl.ds(q * LANES, LANES)]
                    if kparts == 2:
                        cur = cur + pbuf[r, pl.ds(q * LANES, LANES)]
                    rows[1, r, pl.ds(q * LANES, LANES)] = cur * v
                return 0
            lax.fori_loop(0, CH, scale_row, 0)
            pltpu.sync_copy(rows.at[1], hs_s.at[pl.ds(s * rpt + i * CH, CH)])
            return 0
        lax.fori_loop(0, nblk, hscopy, 0)
        for i in range(nblk):
            pltpu.make_async_copy(
                rows.at[0], acc.at[pl.ds(s * rpt + i * CH, CH)], ssem).wait()
        plsc.subcore_barrier()

        pltpu.sync_copy(row_hbm.at[pl.ds(w * nchunks, nchunks)], ridx)
        pltpu.sync_copy(col_hbm.at[pl.ds(w * nchunks, nchunks)], cidx)

        NB = 3
        LAG = 2
        bufs = [rows.at[0], rows.at[1], pbuf]
        ntail = nchunks % NB
        nloop = nchunks - ntail
        for b in range(min(NB, nchunks)):
            pltpu.async_copy(hs_s.at[ridx.at[b]], bufs[b], sem)

        def ebody(g, _):
            for b in range(NB):
                j = g * NB + b
                pltpu.make_async_copy(
                    hs_s.at[ridx.at[j]], bufs[b], sem).wait()
                pltpu.async_copy(bufs[b], acc.at[cidx.at[j]], ssem,
                                 add=True)
                dj = j - LAG
                bd = (b - LAG) % NB

                @pl.when(dj >= 0)
                def _():
                    pltpu.make_async_copy(
                        bufs[bd], acc.at[cidx.at[dj]], ssem).wait()
                    nj = dj + NB

                    @pl.when(nj < nchunks)
                    def _():
                        pltpu.async_copy(
                            hs_s.at[ridx.at[nj]], bufs[bd], sem)
            return 0
        lax.fori_loop(0, nloop // NB, ebody, 0)
        for t in range(ntail):
            j = nloop + t
            b = j % NB
            pltpu.make_async_copy(hs_s.at[ridx.at[j]], bufs[b], sem).wait()
            pltpu.async_copy(bufs[b], acc.at[cidx.at[j]], ssem, add=True)
            dj = j - LAG
            if dj >= 0:
                pltpu.make_async_copy(
                    bufs[dj % NB], acc.at[cidx.at[dj]], ssem).wait()
                nj = dj + NB
                if nj < nchunks:
                    pltpu.async_copy(hs_s.at[ridx.at[nj]], bufs[nj % NB], sem)
        for t in range(min(LAG, nchunks)):
            j = nchunks - min(LAG, nchunks) + t
            pltpu.make_async_copy(
                bufs[j % NB], acc.at[cidx.at[j]], ssem).wait()

        plsc.subcore_barrier()

        for i in range(nblk):
            b = i % 2
            pltpu.sync_copy(acc.at[pl.ds(s * rpt + i * CH, CH)], rows.at[b])
            if i >= 1:
                pltpu.make_async_copy(
                    rows.at[(i - 1) % 2],
                    out_hbm.at[c, pl.ds(s * rpt + (i - 1) * CH, CH)],
                    ssem).wait()
            pltpu.async_copy(rows.at[b],
                             out_hbm.at[c, pl.ds(s * rpt + i * CH, CH)], ssem)
        pltpu.make_async_copy(
            rows.at[(nblk - 1) % 2],
            out_hbm.at[c, pl.ds(s * rpt + (nblk - 1) * CH, CH)], ssem).wait()

    return k


def _tc_prep(x, w_feat, b_feat, n_pad):
    n, _ = x.shape
    dh = w_feat.shape[1]

    def body(x_ref, wf_ref, bf_ref, h_ref):
        h = jnp.maximum(
            jnp.dot(x_ref[...], wf_ref[...],
                    preferred_element_type=jnp.float32) + bf_ref[...][None, :],
            0.0)
        h_ref[:n, :] = h
        h_ref[n:, :] = jnp.zeros((n_pad - n, dh), jnp.float32)

    return pl.pallas_call(
        body,
        out_shape=jax.ShapeDtypeStruct((n_pad, dh), jnp.float32),
    )(x, w_feat, b_feat)


def _tc_layer(h_pad, parts1, parts2, dinv, wl, bl, gl, betal, n_pad,
              final_w=None, final_b=None):
    n = dinv.shape[0]
    dh = h_pad.shape[1]
    final = final_w is not None

    def body(h_ref, p1_ref, p2_ref, dinv_ref, wl_ref, bl_ref, gl_ref,
             betal_ref, *rest):
        if final:
            wc_ref, bc_ref, out_ref = rest
        else:
            (hn_ref,) = rest
        dinv = dinv_ref[...]
        agg1 = dinv * (p1_ref[0, :n, :] + p1_ref[1, :n, :])
        agg2 = dinv * (p2_ref[0, :n, :] + p2_ref[1, :n, :])
        z = (jnp.dot(h_ref[:n, :], wl_ref[:dh, :],
                     preferred_element_type=jnp.float32)
             + jnp.dot(agg1, wl_ref[dh:2 * dh, :],
                       preferred_element_type=jnp.float32)
             + jnp.dot(agg2, wl_ref[2 * dh:, :],
                       preferred_element_type=jnp.float32)
             + bl_ref[...][None, :])
        hn = jnp.maximum(gl_ref[...][None, :] * z * _BN_SCALE
                         + betal_ref[...][None, :], 0.0)
        if final:
            out_ref[...] = jnp.dot(hn, wc_ref[...],
                                   preferred_element_type=jnp.float32) \
                + bc_ref[...][None, :]
        else:
            hn_ref[:n, :] = hn
            hn_ref[n:, :] = jnp.zeros((n_pad - n, dh), jnp.float32)

    if final:
        out_shape = jax.ShapeDtypeStruct((n, final_w.shape[1]), jnp.float32)
        return pl.pallas_call(body, out_shape=out_shape)(
            h_pad, parts1, parts2, dinv, wl, bl, gl, betal, final_w, final_b)
    out_shape = jax.ShapeDtypeStruct((n_pad, dh), jnp.float32)
    return pl.pallas_call(body, out_shape=out_shape)(
        h_pad, parts1, parts2, dinv, wl, bl, gl, betal)


def kernel(x, edge_index, W_feat, b_feat, W1, b1, g1, beta1, W2, b2, g2,
           beta2, Wc, bc):
    n, _ = x.shape
    e = edge_index.shape[1]
    dh = W_feat.shape[1]

    rpt = -(-n // NS)
    rpt = -(-rpt // CH) * CH
    n_pad = rpt * NS
    per_w = -(-e // (NC * NS))
    ew = -(-per_w // (8 * CH)) * (8 * CH)
    e_pad = ew * NC * NS
    nchunks = ew // CH

    row = edge_index[0]
    col = edge_index[1]
    pad = e_pad - e
    if pad:
        fill = jnp.full((pad,), n, jnp.int32)
        row_p = jnp.concatenate([row, fill])
        col_p = jnp.concatenate([col, fill])
    else:
        row_p, col_p = row, col
    row2d = row_p.reshape(NC * NS * nchunks, CH)
    col2d = col_p.reshape(NC * NS * nchunks, CH)

    dd = _sc_deg(n_pad, nchunks, n)(row2d).reshape(2, n_pad)
    dinv_pad, dinv2_pad = dd[0], dd[1]
    dinv = dinv_pad[:n].reshape(n, 1)
    h = _tc_prep(x, W_feat, b_feat, n_pad)

    hop1 = _sc_hop(n_pad, nchunks, dh, 1)
    hop2 = _sc_hop(n_pad, nchunks, dh, 2)
    for li, (wl, bl, gl, betal) in enumerate([(W1, b1, g1, beta1),
                                              (W2, b2, g2, beta2)]):
        parts1 = hop1(h.reshape(1, n_pad, dh), dinv_pad, row2d, col2d)
        parts2 = hop2(parts1, dinv2_pad, row2d, col2d)
        if li == 0:
            h = _tc_layer(h, parts1, parts2, dinv, wl, bl, gl, betal, n_pad)
        else:
            return _tc_layer(h, parts1, parts2, dinv, wl, bl, gl, betal,
                             n_pad, final_w=Wc, final_b=bc)
